# baseline (speedup 1.0000x reference)
import jax
import jax.numpy as jnp
from jax import lax
from jax.experimental import pallas as pl
from jax.experimental.pallas import tpu as pltpu

_ROW_TILE = 3200


def _fused_heads_kernel(x_ref, w_ref, b_ref, o_ref):
    o_ref[...] = (
        lax.dot_general(
            w_ref[...],
            x_ref[...],
            (((1,), (1,)), ((), ())),
            preferred_element_type=jnp.float32,
        )
        + b_ref[...]
    )


def kernel(local_features, W_coords, b_coords, W_pres, b_pres):
    B, C, R, D = local_features.shape
    M = B * C * R
    x = local_features.reshape(M, D)
    w = jnp.concatenate([W_coords, W_pres], axis=0)
    b = jnp.concatenate([b_coords, b_pres], axis=0).reshape(5, 1)

    tile = _ROW_TILE
    grid = (M // tile,)

    out = pl.pallas_call(
        _fused_heads_kernel,
        grid=grid,
        in_specs=[
            pl.BlockSpec((tile, D), lambda i: (i, 0)),
            pl.BlockSpec((5, D), lambda i: (0, 0)),
            pl.BlockSpec((5, 1), lambda i: (0, 0)),
        ],
        out_specs=pl.BlockSpec((5, tile), lambda i: (0, i)),
        out_shape=jax.ShapeDtypeStruct((5, M), jnp.float32),
        compiler_params=pltpu.CompilerParams(
            dimension_semantics=("arbitrary",),
        ),
    )(x, w, b)

    coords = out[:4].T.reshape(B, C, R, 4)
    pres = out[4:].T.reshape(B, C, R, 1)
    return (coords, pres)


# speedup vs baseline: 1.0204x; 1.0204x over previous
import jax
import jax.numpy as jnp
from jax import lax
from jax.experimental import pallas as pl
from jax.experimental.pallas import tpu as pltpu

_ROW_TILE = 6400


def _fused_heads_kernel(x_ref, w_ref, b_ref, o_ref):
    o_ref[...] = (
        lax.dot_general(
            w_ref[...],
            x_ref[...],
            (((1,), (1,)), ((), ())),
            preferred_element_type=jnp.float32,
        )
        + b_ref[...]
    )


def kernel(local_features, W_coords, b_coords, W_pres, b_pres):
    B, C, R, D = local_features.shape
    M = B * C * R
    x = local_features.reshape(M, D)
    w = jnp.concatenate([W_coords, W_pres], axis=0)
    b = jnp.concatenate([b_coords, b_pres], axis=0).reshape(5, 1)

    tile = _ROW_TILE
    grid = (M // tile,)

    out = pl.pallas_call(
        _fused_heads_kernel,
        grid=grid,
        in_specs=[
            pl.BlockSpec((tile, D), lambda i: (i, 0)),
            pl.BlockSpec((5, D), lambda i: (0, 0)),
            pl.BlockSpec((5, 1), lambda i: (0, 0)),
        ],
        out_specs=pl.BlockSpec((5, tile), lambda i: (0, i)),
        out_shape=jax.ShapeDtypeStruct((5, M), jnp.float32),
        compiler_params=pltpu.CompilerParams(
            dimension_semantics=("arbitrary",),
        ),
    )(x, w, b)

    coords = out[:4].T.reshape(B, C, R, 4)
    pres = out[4:].T.reshape(B, C, R, 1)
    return (coords, pres)
